# baseline (speedup 1.0000x reference)
import functools

import jax
import jax.numpy as jnp
from jax import lax
from jax.experimental import pallas as pl
from jax.experimental.pallas import tpu as pltpu
from jax.experimental.pallas import tpu_sc as plsc

N_HEADS = 16
SEQ = 2048
TAB = 255
GLEN = 2 * SEQ - 1
SLOT = 4352
NC = 2
NS = 16
N_TILES = NC * NS
ROWS_PER_TILE = N_HEADS * SEQ // N_TILES
FIRE = 16


def _sc_body(table_hbm, out_hbm, table_v, g8_v, sem):
    cid = lax.axis_index("c")
    sid = lax.axis_index("s")
    wid = sid * NC + cid
    head = wid >> 1
    half = wid & 1

    pltpu.sync_copy(table_hbm, table_v)

    lane = lax.iota(jnp.int32, 16)

    for s in range(8):
        @pl.loop(0, SLOT // 16)
        def _build(c, s=s):
            p = c * 16 + lane
            idx = jnp.clip(p + (s - (SEQ - 128)), 0, TAB - 1) * N_HEADS + head
            vals = plsc.load_gather(table_v, [idx])
            g8_v[pl.ds(s * SLOT + c * 16, 16)] = vals

    row0 = head * SEQ + half * (SEQ // 2)

    def _row_copy(r, k):
        i_local = r * FIRE + k
        i = half * (SEQ // 2) + i_local
        start = (SEQ - 1) - i
        s = (7 - k) & 7
        base = pl.multiple_of(start - s, 8)
        src = g8_v.at[pl.ds(s * SLOT + base, SEQ)]
        dst = out_hbm.at[pl.ds((row0 + i_local) * SEQ, SEQ)]
        return pltpu.make_async_copy(src, dst, sem)

    @pl.loop(0, ROWS_PER_TILE // FIRE)
    def _rows(r):
        for k in range(FIRE):
            _row_copy(r, k).start()

        @pl.when(r > 0)
        def _():
            for k in range(FIRE):
                _row_copy(r - 1, k).wait()

    for k in range(FIRE):
        _row_copy(ROWS_PER_TILE // FIRE - 1, k).wait()


@jax.jit
def _relative_pos_bias_sc(bias_table):
    mesh = plsc.VectorSubcoreMesh(core_axis_name="c", subcore_axis_name="s")
    fn = pl.kernel(
        _sc_body,
        out_type=jax.ShapeDtypeStruct((N_HEADS * SEQ * SEQ,), jnp.float32),
        mesh=mesh,
        scratch_types=[
            pltpu.VMEM((TAB * N_HEADS,), jnp.float32),
            pltpu.VMEM((8 * SLOT,), jnp.float32),
            pltpu.SemaphoreType.DMA,
        ],
        compiler_params=pltpu.CompilerParams(needs_layout_passes=False),
    )
    out_flat = fn(bias_table.reshape(-1))
    return out_flat.reshape(1, N_HEADS, SEQ, SEQ)


def kernel(seq_len, bias_table):
    del seq_len
    return _relative_pos_bias_sc(bias_table)


# speedup vs baseline: 2.1548x; 2.1548x over previous
import functools

import jax
import jax.numpy as jnp
from jax import lax
from jax.experimental import pallas as pl
from jax.experimental.pallas import tpu as pltpu
from jax.experimental.pallas import tpu_sc as plsc

N_HEADS = 16
SEQ = 2048
TAB = 255
GPAD = 2 * SEQ
NC = 2
NS = 16
BI = 128


def _sc_gather_body(table_hbm, g_hbm, table_v, row_v, sem):
    cid = lax.axis_index("c")
    sid = lax.axis_index("s")
    wid = sid * NC + cid
    head = wid >> 1
    half = wid & 1

    pltpu.sync_copy(table_hbm, table_v)
    lane = lax.iota(jnp.int32, 16)
    p0 = half * (GPAD // 2)

    @pl.loop(0, GPAD // 2 // 16)
    def _build(c):
        p = p0 + c * 16 + lane
        idx = jnp.clip(p - (SEQ - 127), 0, TAB - 1) * N_HEADS + head
        row_v[pl.ds(c * 16, 16)] = plsc.load_gather(table_v, [idx])

    pltpu.sync_copy(row_v, g_hbm.at[pl.ds(head * GPAD + p0, GPAD // 2)])


def _sc_gather(table_flat):
    mesh = plsc.VectorSubcoreMesh(core_axis_name="c", subcore_axis_name="s")
    fn = pl.kernel(
        _sc_gather_body,
        out_type=jax.ShapeDtypeStruct((N_HEADS * GPAD,), jnp.float32),
        mesh=mesh,
        scratch_types=[
            pltpu.VMEM((TAB * N_HEADS,), jnp.float32),
            pltpu.VMEM((GPAD // 2,), jnp.float32),
            pltpu.SemaphoreType.DMA,
        ],
        compiler_params=pltpu.CompilerParams(needs_layout_passes=False),
    )
    return fn(table_flat)


def _tc_expand_body(g_ref, out_ref, r_ref):
    bi = pl.program_id(1)

    @pl.when(bi == 0)
    def _build_r():
        a = g_ref[0]
        for s in range(7):
            a = jnp.concatenate([a, pltpu.roll(a, 1 << s, axis=1)], axis=0)
        r_ref[...] = a

    start = pl.multiple_of(SEQ - BI * bi, BI)
    out_ref[0] = r_ref[:, pl.ds(start, SEQ)]


def _tc_expand(g, interpret=False):
    return pl.pallas_call(
        _tc_expand_body,
        grid=(N_HEADS, SEQ // BI),
        in_specs=[pl.BlockSpec((1, 1, GPAD), lambda h, bi: (h, 0, 0))],
        out_specs=pl.BlockSpec((1, BI, SEQ), lambda h, bi: (h, bi, 0)),
        out_shape=jax.ShapeDtypeStruct((N_HEADS, SEQ, SEQ), jnp.float32),
        scratch_shapes=[pltpu.VMEM((BI, GPAD), jnp.float32)],
        interpret=interpret,
    )(g)


@jax.jit
def _relative_pos_bias(bias_table):
    g = _sc_gather(bias_table.reshape(-1))
    out = _tc_expand(g.reshape(N_HEADS, 1, GPAD))
    return out.reshape(1, N_HEADS, SEQ, SEQ)


def kernel(seq_len, bias_table):
    del seq_len
    return _relative_pos_bias(bias_table)


# speedup vs baseline: 3.6689x; 1.7027x over previous
import functools

import jax
import jax.numpy as jnp
from jax import lax
from jax.experimental import pallas as pl
from jax.experimental.pallas import tpu as pltpu
from jax.experimental.pallas import tpu_sc as plsc

N_HEADS = 16
SEQ = 2048
TAB = 255
GPAD = 2 * SEQ
NC = 2
NS = 16
BI = 128


def _sc_gather_body(table_hbm, g_hbm, table_v, row_v, sem):
    cid = lax.axis_index("c")
    sid = lax.axis_index("s")
    wid = sid * NC + cid
    head = wid >> 1
    half = wid & 1

    pltpu.sync_copy(table_hbm, table_v)
    lane = lax.iota(jnp.int32, 16)
    p0 = half * (GPAD // 2)

    @pl.loop(0, GPAD // 2 // 16)
    def _build(c):
        p = p0 + c * 16 + lane
        idx = jnp.clip(p - (SEQ - 127), 0, TAB - 1) * N_HEADS + head
        row_v[pl.ds(c * 16, 16)] = plsc.load_gather(table_v, [idx])

    pltpu.sync_copy(row_v, g_hbm.at[pl.ds(head * GPAD + p0, GPAD // 2)])


def _sc_gather(table_flat):
    mesh = plsc.VectorSubcoreMesh(core_axis_name="c", subcore_axis_name="s")
    fn = pl.kernel(
        _sc_gather_body,
        out_type=jax.ShapeDtypeStruct((N_HEADS * GPAD,), jnp.float32),
        mesh=mesh,
        scratch_types=[
            pltpu.VMEM((TAB * N_HEADS,), jnp.float32),
            pltpu.VMEM((GPAD // 2,), jnp.float32),
            pltpu.SemaphoreType.DMA,
        ],
        compiler_params=pltpu.CompilerParams(needs_layout_passes=False),
    )
    return fn(table_flat)


def _tc_expand_body(g_ref, out_ref, r_ref, sem_ref):
    h = pl.program_id(0)
    par = lax.rem(h, 2)

    def copies(hh, buf):
        out = []
        for bi in range(SEQ // BI):
            src = r_ref.at[buf, :, pl.ds(SEQ - BI * bi, SEQ)]
            dst = out_ref.at[hh, pl.ds(bi * BI, BI), :]
            out.append(pltpu.make_async_copy(src, dst, sem_ref.at[buf]))
        return out

    @pl.when(h >= 2)
    def _drain_prev():
        for c in copies(h - 2, par):
            c.wait()

    a = g_ref[0]
    for s in range(7):
        a = jnp.concatenate([a, pltpu.roll(a, 1 << s, axis=1)], axis=0)
    r_ref[par] = a

    for c in copies(h, par):
        c.start()

    @pl.when(h == N_HEADS - 1)
    def _drain_tail():
        for c in copies(h - 1, 1 - par):
            c.wait()
        for c in copies(h, par):
            c.wait()


def _tc_expand(g, interpret=False):
    return pl.pallas_call(
        _tc_expand_body,
        grid=(N_HEADS,),
        in_specs=[pl.BlockSpec((1, 1, GPAD), lambda h: (h, 0, 0))],
        out_specs=pl.BlockSpec(memory_space=pl.ANY),
        out_shape=jax.ShapeDtypeStruct((N_HEADS, SEQ, SEQ), jnp.float32),
        scratch_shapes=[
            pltpu.VMEM((2, BI, GPAD), jnp.float32),
            pltpu.SemaphoreType.DMA((2,)),
        ],
        interpret=interpret,
    )(g)


@jax.jit
def _relative_pos_bias(bias_table):
    g = _sc_gather(bias_table.reshape(-1))
    out = _tc_expand(g.reshape(N_HEADS, 1, GPAD))
    return out.reshape(1, N_HEADS, SEQ, SEQ)


def kernel(seq_len, bias_table):
    del seq_len
    return _relative_pos_bias(bias_table)
